# Initial kernel scaffold; baseline (speedup 1.0000x reference)
import jax
import jax.numpy as jnp
from jax.experimental import pallas as pl


def kernel(W, X):
    raise NotImplementedError("write your pallas kernel here")


# speedup vs baseline: 25.4319x; 25.4319x over previous
import jax
import jax.numpy as jnp
from jax import lax
from jax.experimental import pallas as pl
from jax.experimental.pallas import tpu as pltpu
from jax.experimental.pallas import tpu_sc as plsc

_ROWS = 8192
_COLS = 2048
_NB = 65536
_NW = 32
_RPW = _ROWS // _NW
_CHUNK = 8
_NCHUNKS = _RPW // _CHUNK
_K = (_ROWS * _COLS) // 2


def _colsq(xf):
    def body(x_ref, o_ref):
        i = pl.program_id(0)
        part = jnp.sum(x_ref[...] * x_ref[...], axis=0, keepdims=True)

        @pl.when(i == 0)
        def _():
            o_ref[...] = part

        @pl.when(i != 0)
        def _():
            o_ref[...] += part

    return pl.pallas_call(
        body,
        grid=(32,),
        in_specs=[pl.BlockSpec((256, _COLS), lambda i: (i, 0))],
        out_specs=pl.BlockSpec((1, _COLS), lambda i: (0, 0)),
        out_shape=jax.ShapeDtypeStruct((1, _COLS), jnp.float32),
    )(xf)


def _mask_pass(w, ss2d, thr):
    def body(w_ref, ss_ref, t_ref, o_ref):
        q = (w_ref[...] * w_ref[...]) * ss_ref[...]
        o_ref[...] = q >= t_ref[...]

    return pl.pallas_call(
        body,
        grid=(32,),
        in_specs=[
            pl.BlockSpec((256, _COLS), lambda i: (i, 0)),
            pl.BlockSpec((1, _COLS), lambda i: (0, 0)),
            pl.BlockSpec((1, 1), lambda i: (0, 0)),
        ],
        out_specs=pl.BlockSpec((256, _COLS), lambda i: (i, 0)),
        out_shape=jax.ShapeDtypeStruct((_ROWS, _COLS), jnp.bool_),
    )(w, ss2d, thr.reshape(1, 1))


def _hist_body(low_pass, *refs):
    if low_pass:
        w_hbm, ss_hbm, sel_hbm, out_hbm, ssv, wbuf, hist, selv = refs
    else:
        w_hbm, ss_hbm, out_hbm, ssv, wbuf, hist = refs

    cid = lax.axis_index("c")
    sid = lax.axis_index("s")
    wid = sid * 2 + cid

    zeros16 = jnp.zeros((16,), jnp.int32)

    def _zero(i, carry):
        hist[pl.ds(i * 16, 16)] = zeros16
        return carry

    lax.fori_loop(0, _NB // 16, _zero, 0)

    pltpu.sync_copy(ss_hbm, ssv)
    if low_pass:
        pltpu.sync_copy(sel_hbm, selv)
        sel = selv[...]

    ones16 = jnp.ones((16,), jnp.int32)
    shift16 = jnp.full((16,), 16, jnp.int32)
    mask16 = jnp.full((16,), 0xFFFF, jnp.int32)

    def _chunk(ci, carry):
        row0 = wid * _RPW + ci * _CHUNK
        pltpu.sync_copy(w_hbm.at[pl.ds(row0, _CHUNK)], wbuf)

        def _group(g, c2):
            col = g * 16
            sv = ssv[pl.ds(col, 16)]
            for r in range(_CHUNK):
                w = wbuf[r, pl.ds(col, 16)]
                q = (w * w) * sv
                bits = lax.bitcast_convert_type(q, jnp.int32)
                hi = lax.shift_right_logical(bits, shift16)
                if low_pass:
                    lo = jnp.bitwise_and(bits, mask16)
                    plsc.addupdate_scatter(hist, [lo], ones16, mask=hi == sel)
                else:
                    plsc.addupdate_scatter(hist, [hi], ones16)
            return c2

        lax.fori_loop(0, _COLS // 16, _group, 0)
        return carry

    lax.fori_loop(0, _NCHUNKS, _chunk, 0)
    pltpu.sync_copy(hist, out_hbm.at[wid])


def _run_hist(w, ss, sel=None):
    low_pass = sel is not None
    mesh = plsc.VectorSubcoreMesh(core_axis_name="c", subcore_axis_name="s")
    scratch = [
        pltpu.VMEM((_COLS,), jnp.float32),
        pltpu.VMEM((_CHUNK, _COLS), jnp.float32),
        pltpu.VMEM((_NB,), jnp.int32),
    ]
    if low_pass:
        scratch.append(pltpu.VMEM((16,), jnp.int32))

    def body(*refs):
        _hist_body(low_pass, *refs)

    f = pl.kernel(
        body,
        out_type=jax.ShapeDtypeStruct((_NW, _NB), jnp.int32),
        mesh=mesh,
        scratch_types=scratch,
        compiler_params=pltpu.CompilerParams(needs_layout_passes=False),
    )
    args = (w, ss, sel) if low_pass else (w, ss)
    return f(*args)


def _pick(hist, rank):
    cnt_ge = jnp.cumsum(hist[::-1])[::-1]
    idx = jnp.arange(_NB, dtype=jnp.int32)
    b = jnp.max(jnp.where(cnt_ge >= rank, idx, -1)).astype(jnp.int32)
    cnt_pad = jnp.concatenate([cnt_ge, jnp.zeros((1,), cnt_ge.dtype)])
    above = cnt_pad[b + 1]
    return b, rank - above


def kernel(W, X):
    xf = X.reshape(_ROWS, _COLS)
    ss2d = _colsq(xf)
    ss = ss2d.reshape(_COLS)

    h1 = jnp.sum(_run_hist(W, ss), axis=0)
    b1, r1 = _pick(h1, jnp.int32(_K))

    sel = jnp.full((16,), b1, jnp.int32)
    h2 = jnp.sum(_run_hist(W, ss, sel), axis=0)
    b2, _ = _pick(h2, r1)

    thr_bits = (b1 << 16) | b2
    thr = lax.bitcast_convert_type(thr_bits, jnp.float32)
    return _mask_pass(W, ss2d, thr)
